# baseline (speedup 1.0000x reference)
import functools

import jax
import jax.numpy as jnp
from jax import lax
from jax.experimental import pallas as pl
from jax.experimental.pallas import tpu as pltpu
from jax.experimental.pallas import tpu_sc as plsc

_ROWS = 8192
_DIM = 2048
_NC = 2
_NS = 16
_NW = _NC * _NS
_ROWS_PER_W = _ROWS // _NW
_CHUNK = 8
_NCHUNK = _ROWS_PER_W // _CHUNK
_LANES = 16
_NBLK = _DIM // _LANES
_NBUF = 3
_GR = _CHUNK // 2
_U = 4
_NL = (_NBLK - 2) // _U


def _permute_body(x_hbm, perm_hbm, out_hbm, perm_v, inv_v,
                  xb0, xb1, xb2, ob0, ob1, ob2,
                  is0, is1, is2, os0, os1, os2):
    wid = lax.axis_index("s") * _NC + lax.axis_index("c")
    base = wid * _ROWS_PER_W

    pltpu.sync_copy(perm_hbm, perm_v)

    lane = lax.iota(jnp.int32, _LANES)

    def inv_body(b, _):
        col = b * _LANES
        plsc.store_scatter(inv_v, [perm_v[pl.ds(col, _LANES)]], col + lane)
        return 0

    lax.fori_loop(0, _NBLK, inv_body, 0)

    xbufs, obufs = (xb0, xb1, xb2), (ob0, ob1, ob2)
    isems, osems = (is0, is1, is2), (os0, os1, os2)

    def in_copy(c, s):
        return pltpu.make_async_copy(
            x_hbm.at[pl.ds(base + c * _CHUNK, _CHUNK)], xbufs[s], isems[s])

    def out_copy(c, s):
        return pltpu.make_async_copy(
            obufs[s], out_hbm.at[pl.ds(base + c * _CHUNK, _CHUNK)], osems[s])

    def compute(s):
        xbuf, obuf = xbufs[s], obufs[s]

        def load_blocks(col):
            return tuple(xbuf[r, pl.ds(col, _LANES)] for r in range(_CHUNK))

        def store_blocks(inv, sv):
            for r in range(_CHUNK):
                plsc.store_scatter(
                    obuf, [jnp.full((_LANES,), r, jnp.int32), inv], sv[r])

        inv0 = inv_v[pl.ds(0, _LANES)]
        sv0 = load_blocks(0)

        def body(g, carry):
            inv_p, sv_p = carry
            for u in range(_U):
                j = _U * g + 1 + u
                col = j * _LANES
                sv = load_blocks(col)
                inv = inv_v[pl.ds(col, _LANES)]
                store_blocks(inv_p, sv_p)
                inv_p, sv_p = inv, sv
            return inv_p, sv_p

        inv_p, sv_p = lax.fori_loop(0, _NL, body, (inv0, sv0))
        for j in range(_NL * _U + 1, _NBLK):
            col = j * _LANES
            sv = load_blocks(col)
            inv = inv_v[pl.ds(col, _LANES)]
            store_blocks(inv_p, sv_p)
            inv_p, sv_p = inv, sv
        store_blocks(inv_p, sv_p)

    for c in range(_NBUF - 1):
        in_copy(c, c).start()
    for c in range(_NCHUNK):
        s = c % _NBUF
        in_copy(c, s).wait()
        if c >= _NBUF:
            out_copy(c - _NBUF, s).wait()
        if c + _NBUF - 1 < _NCHUNK:
            in_copy(c + _NBUF - 1, (c + _NBUF - 1) % _NBUF).start()
        compute(s)
        out_copy(c, s).start()
    for c in range(_NCHUNK - _NBUF, _NCHUNK):
        out_copy(c, c % _NBUF).wait()


_permute = functools.partial(
    pl.kernel,
    out_type=jax.ShapeDtypeStruct((_ROWS, _DIM), jnp.float32),
    mesh=plsc.VectorSubcoreMesh(core_axis_name="c", subcore_axis_name="s"),
    scratch_types=[
        pltpu.VMEM((_DIM,), jnp.int32),
        pltpu.VMEM((_DIM,), jnp.int32),
        pltpu.VMEM((_CHUNK, _DIM), jnp.float32),
        pltpu.VMEM((_CHUNK, _DIM), jnp.float32),
        pltpu.VMEM((_CHUNK, _DIM), jnp.float32),
        pltpu.VMEM((_CHUNK, _DIM), jnp.float32),
        pltpu.VMEM((_CHUNK, _DIM), jnp.float32),
        pltpu.VMEM((_CHUNK, _DIM), jnp.float32),
        pltpu.SemaphoreType.DMA,
        pltpu.SemaphoreType.DMA,
        pltpu.SemaphoreType.DMA,
        pltpu.SemaphoreType.DMA,
        pltpu.SemaphoreType.DMA,
        pltpu.SemaphoreType.DMA,
    ],
    compiler_params=pltpu.CompilerParams(needs_layout_passes=False),
)(_permute_body)


@jax.jit
def kernel(x, perm):
    return _permute(x, perm.astype(jnp.int32))


# speedup vs baseline: 1.1374x; 1.1374x over previous
import functools

import jax
import jax.numpy as jnp
from jax import lax
from jax.experimental import pallas as pl
from jax.experimental.pallas import tpu as pltpu
from jax.experimental.pallas import tpu_sc as plsc

_ROWS = 8192
_DIM = 2048
_NC = 2
_NS = 16
_NW = _NC * _NS
_ROWS_PER_W = _ROWS // _NW
_CHUNK = 8
_NCHUNK = _ROWS_PER_W // _CHUNK
_LANES = 16
_NBLK = _DIM // _LANES
_NBUF = 3
_GR = 6
_SR = _CHUNK - _GR
_U = 4
_NL = (_NBLK - 2) // _U


def _permute_body(x_hbm, perm_hbm, out_hbm, perm_v, inv_v,
                  xb0, xb1, xb2, ob0, ob1, ob2,
                  is0, is1, is2, os0, os1, os2):
    wid = lax.axis_index("s") * _NC + lax.axis_index("c")
    base = wid * _ROWS_PER_W

    pltpu.sync_copy(perm_hbm, perm_v)

    lane = lax.iota(jnp.int32, _LANES)

    def inv_body(b, _):
        col = b * _LANES
        plsc.store_scatter(inv_v, [perm_v[pl.ds(col, _LANES)]], col + lane)
        return 0

    lax.fori_loop(0, _NBLK, inv_body, 0)

    xbufs, obufs = (xb0, xb1, xb2), (ob0, ob1, ob2)
    isems, osems = (is0, is1, is2), (os0, os1, os2)

    def in_copy(c, s):
        return pltpu.make_async_copy(
            x_hbm.at[pl.ds(base + c * _CHUNK, _CHUNK)], xbufs[s], isems[s])

    def out_copy(c, s):
        return pltpu.make_async_copy(
            obufs[s], out_hbm.at[pl.ds(base + c * _CHUNK, _CHUNK)], osems[s])

    def compute(s):
        xbuf, obuf = xbufs[s], obufs[s]

        def load_blocks(col, idx):
            gv = tuple(
                plsc.load_gather(xbuf, [jnp.full((_LANES,), r, jnp.int32), idx])
                for r in range(_GR))
            sv = tuple(xbuf[_GR + r, pl.ds(col, _LANES)] for r in range(_SR))
            return gv, sv

        def store_blocks(col, inv, gv, sv):
            for r in range(_GR):
                obuf[r, pl.ds(col, _LANES)] = gv[r]
            for r in range(_SR):
                plsc.store_scatter(
                    obuf, [jnp.full((_LANES,), _GR + r, jnp.int32), inv], sv[r])

        inv0 = inv_v[pl.ds(0, _LANES)]
        gv0, sv0 = load_blocks(0, perm_v[pl.ds(0, _LANES)])
        idx1 = perm_v[pl.ds(_LANES, _LANES)]

        def body(g, carry):
            idx, inv_p, gv_p, sv_p = carry
            for u in range(_U):
                j = _U * g + 1 + u
                col = j * _LANES
                gv, sv = load_blocks(col, idx)
                inv = inv_v[pl.ds(col, _LANES)]
                idx = perm_v[pl.ds((j + 1) * _LANES, _LANES)]
                store_blocks(col - _LANES, inv_p, gv_p, sv_p)
                inv_p, gv_p, sv_p = inv, gv, sv
            return idx, inv_p, gv_p, sv_p

        idx, inv_p, gv_p, sv_p = lax.fori_loop(
            0, _NL, body, (idx1, inv0, gv0, sv0))
        for j in range(_NL * _U + 1, _NBLK):
            col = j * _LANES
            gv, sv = load_blocks(col, idx)
            inv = inv_v[pl.ds(col, _LANES)]
            if j + 1 < _NBLK:
                idx = perm_v[pl.ds((j + 1) * _LANES, _LANES)]
            store_blocks(col - _LANES, inv_p, gv_p, sv_p)
            inv_p, gv_p, sv_p = inv, gv, sv
        store_blocks((_NBLK - 1) * _LANES, inv_p, gv_p, sv_p)

    for c in range(_NBUF - 1):
        in_copy(c, c).start()
    for c in range(_NCHUNK):
        s = c % _NBUF
        in_copy(c, s).wait()
        if c >= _NBUF:
            out_copy(c - _NBUF, s).wait()
        if c + _NBUF - 1 < _NCHUNK:
            in_copy(c + _NBUF - 1, (c + _NBUF - 1) % _NBUF).start()
        compute(s)
        out_copy(c, s).start()
    for c in range(_NCHUNK - _NBUF, _NCHUNK):
        out_copy(c, c % _NBUF).wait()


_permute = functools.partial(
    pl.kernel,
    out_type=jax.ShapeDtypeStruct((_ROWS, _DIM), jnp.float32),
    mesh=plsc.VectorSubcoreMesh(core_axis_name="c", subcore_axis_name="s"),
    scratch_types=[
        pltpu.VMEM((_DIM,), jnp.int32),
        pltpu.VMEM((_DIM,), jnp.int32),
        pltpu.VMEM((_CHUNK, _DIM), jnp.float32),
        pltpu.VMEM((_CHUNK, _DIM), jnp.float32),
        pltpu.VMEM((_CHUNK, _DIM), jnp.float32),
        pltpu.VMEM((_CHUNK, _DIM), jnp.float32),
        pltpu.VMEM((_CHUNK, _DIM), jnp.float32),
        pltpu.VMEM((_CHUNK, _DIM), jnp.float32),
        pltpu.SemaphoreType.DMA,
        pltpu.SemaphoreType.DMA,
        pltpu.SemaphoreType.DMA,
        pltpu.SemaphoreType.DMA,
        pltpu.SemaphoreType.DMA,
        pltpu.SemaphoreType.DMA,
    ],
    compiler_params=pltpu.CompilerParams(needs_layout_passes=False),
)(_permute_body)


@jax.jit
def kernel(x, perm):
    return _permute(x, perm.astype(jnp.int32))


# speedup vs baseline: 1.2132x; 1.0666x over previous
import functools

import jax
import jax.numpy as jnp
from jax import lax
from jax.experimental import pallas as pl
from jax.experimental.pallas import tpu as pltpu
from jax.experimental.pallas import tpu_sc as plsc

_ROWS = 8192
_DIM = 2048
_NC = 2
_NS = 16
_NW = _NC * _NS
_ROWS_PER_W = _ROWS // _NW
_CHUNK = 8
_NCHUNK = _ROWS_PER_W // _CHUNK
_LANES = 16
_NBLK = _DIM // _LANES
_NBUF = 3
_GR = _CHUNK // 2
_U = 4
_NL = (_NBLK - 2) // _U


def _permute_body(x_hbm, perm_hbm, out_hbm, perm_v, inv_v,
                  xb0, xb1, xb2, ob0, ob1, ob2,
                  is0, is1, is2, os0, os1, os2):
    wid = lax.axis_index("s") * _NC + lax.axis_index("c")
    base = wid * _ROWS_PER_W

    pltpu.sync_copy(perm_hbm, perm_v)

    lane = lax.iota(jnp.int32, _LANES)

    def inv_body(b, _):
        col = b * _LANES
        plsc.store_scatter(inv_v, [perm_v[pl.ds(col, _LANES)]], col + lane)
        return 0

    lax.fori_loop(0, _NBLK, inv_body, 0)

    xbufs, obufs = (xb0, xb1, xb2), (ob0, ob1, ob2)
    isems, osems = (is0, is1, is2), (os0, os1, os2)

    def in_copy(c, s):
        return pltpu.make_async_copy(
            x_hbm.at[pl.ds(base + c * _CHUNK, _CHUNK)], xbufs[s], isems[s])

    def out_copy(c, s):
        return pltpu.make_async_copy(
            obufs[s], out_hbm.at[pl.ds(base + c * _CHUNK, _CHUNK)], osems[s])

    def compute(s):
        xbuf, obuf = xbufs[s], obufs[s]

        def load_blocks(col, idx):
            gv = tuple(
                plsc.load_gather(xbuf, [jnp.full((_LANES,), r, jnp.int32), idx])
                for r in range(_GR))
            sv = tuple(xbuf[_GR + r, pl.ds(col, _LANES)] for r in range(_GR))
            return gv, sv

        def store_blocks(col, inv, gv, sv):
            for r in range(_GR):
                obuf[r, pl.ds(col, _LANES)] = gv[r]
            for r in range(_GR):
                plsc.store_scatter(
                    obuf, [jnp.full((_LANES,), _GR + r, jnp.int32), inv], sv[r])

        inv0 = inv_v[pl.ds(0, _LANES)]
        gv0, sv0 = load_blocks(0, perm_v[pl.ds(0, _LANES)])
        idx1 = perm_v[pl.ds(_LANES, _LANES)]

        def body(g, carry):
            idx, inv_p, gv_p, sv_p = carry
            for u in range(_U):
                j = _U * g + 1 + u
                col = j * _LANES
                gv, sv = load_blocks(col, idx)
                inv = inv_v[pl.ds(col, _LANES)]
                idx = perm_v[pl.ds((j + 1) * _LANES, _LANES)]
                store_blocks(col - _LANES, inv_p, gv_p, sv_p)
                inv_p, gv_p, sv_p = inv, gv, sv
            return idx, inv_p, gv_p, sv_p

        idx, inv_p, gv_p, sv_p = lax.fori_loop(
            0, _NL, body, (idx1, inv0, gv0, sv0))
        for j in range(_NL * _U + 1, _NBLK):
            col = j * _LANES
            gv, sv = load_blocks(col, idx)
            inv = inv_v[pl.ds(col, _LANES)]
            if j + 1 < _NBLK:
                idx = perm_v[pl.ds((j + 1) * _LANES, _LANES)]
            store_blocks(col - _LANES, inv_p, gv_p, sv_p)
            inv_p, gv_p, sv_p = inv, gv, sv
        store_blocks((_NBLK - 1) * _LANES, inv_p, gv_p, sv_p)

    for c in range(_NBUF - 1):
        in_copy(c, c).start()
    for c in range(_NCHUNK):
        s = c % _NBUF
        in_copy(c, s).wait()
        if c >= _NBUF:
            out_copy(c - _NBUF, s).wait()
        if c + _NBUF - 1 < _NCHUNK:
            in_copy(c + _NBUF - 1, (c + _NBUF - 1) % _NBUF).start()
        compute(s)
        out_copy(c, s).start()
    for c in range(_NCHUNK - _NBUF, _NCHUNK):
        out_copy(c, c % _NBUF).wait()


_permute = functools.partial(
    pl.kernel,
    out_type=jax.ShapeDtypeStruct((_ROWS, _DIM), jnp.float32),
    mesh=plsc.VectorSubcoreMesh(core_axis_name="c", subcore_axis_name="s"),
    scratch_types=[
        pltpu.VMEM((_DIM,), jnp.int32),
        pltpu.VMEM((_DIM,), jnp.int32),
        pltpu.VMEM((_CHUNK, _DIM), jnp.float32),
        pltpu.VMEM((_CHUNK, _DIM), jnp.float32),
        pltpu.VMEM((_CHUNK, _DIM), jnp.float32),
        pltpu.VMEM((_CHUNK, _DIM), jnp.float32),
        pltpu.VMEM((_CHUNK, _DIM), jnp.float32),
        pltpu.VMEM((_CHUNK, _DIM), jnp.float32),
        pltpu.SemaphoreType.DMA,
        pltpu.SemaphoreType.DMA,
        pltpu.SemaphoreType.DMA,
        pltpu.SemaphoreType.DMA,
        pltpu.SemaphoreType.DMA,
        pltpu.SemaphoreType.DMA,
    ],
    compiler_params=pltpu.CompilerParams(needs_layout_passes=False),
)(_permute_body)


@jax.jit
def kernel(x, perm):
    return _permute(x, perm.astype(jnp.int32))
